# baseline (speedup 1.0000x reference)
import numpy as np
import jax
import jax.numpy as jnp
from jax import lax
from jax.experimental import pallas as pl
from jax.experimental.pallas import tpu as pltpu
from jax.experimental.pallas import tpu_sc as plsc

N = 32768
NBINS = 768
NB2 = 2 * NBINS
NC, NS = 2, 16
NW = NC * NS
SHARD = N // NW
KV = SHARD // 16
COLS = 128
NCOLBLK = NB2 // COLS


def _i16(v):
  return jnp.full((16,), v, jnp.int32)


def _sc_body(coords, win0, cin0, win1, cin1, iw0, dl0, iw1, dl1, coorsb,
             cbuf, cbufB, w0v, w1v, r0v, r1v, d0v, d1v, c0v, c1v, c4v, hv,
             tmp, ptmp, basev, totv, semA, semB, semO, sh_hist, sh_pref):
  c = lax.axis_index("c")
  s = lax.axis_index("s")
  wid = c * NS + s
  mirror = (1 - c) * NS + s
  iota = lax.iota(jnp.int32, 16)
  zero16 = jnp.zeros((16,), jnp.int32)

  cp_own = pltpu.async_copy(coords.at[:, pl.ds(wid * SHARD, SHARD)],
                            cbuf, semA)
  cp_mir = pltpu.async_copy(coords.at[:, pl.ds(mirror * SHARD, SHARD)],
                            cbufB, semB)
  pending = []

  def run_pass(shard, is_own, buf, cp):
    def zb(j, _):
      plsc.store_scatter(hv, [zero16, j * 16 + iota], zero16)
      return 0
    lax.fori_loop(0, NB2 // 16, zb, 0)

    cp.wait()

    def kb(k, _):
      rows = k * 16 + iota
      b = plsc.load_gather(buf, [_i16(0), rows])
      z = plsc.load_gather(buf, [_i16(1), rows])
      y = plsc.load_gather(buf, [_i16(2), rows])
      x = plsc.load_gather(buf, [_i16(3), rows])
      if is_own:
        plsc.store_scatter(c4v, [_i16(0), rows], b)
        plsc.store_scatter(c4v, [_i16(1), rows], z)
        plsc.store_scatter(c4v, [_i16(2), rows], y)
        plsc.store_scatter(c4v, [_i16(3), rows], x)
      for off, boff, wv, rv, cv in ((8, 0, w0v, r0v, c0v),
                                    (4, NBINS, w1v, r1v, c1v)):
        sz = z + off
        sy = y + off
        sx = x + off
        w = (b * 48 + jnp.right_shift(sx, 3) * 12
             + jnp.right_shift(sy, 3) * 3 + jnp.right_shift(sz, 3))
        cnt, last = plsc.scan_count(w)
        g = plsc.load_gather(hv, [zero16, w + boff])
        plsc.store_scatter(hv, [zero16, w + boff], g + cnt, mask=last)
        if is_own:
          plsc.store_scatter(wv, [rows], w)
          plsc.store_scatter(rv, [rows], g + cnt - 1)
          plsc.store_scatter(cv, [_i16(0), rows], jnp.bitwise_and(sz, 7))
          plsc.store_scatter(cv, [_i16(1), rows], jnp.bitwise_and(sy, 7))
          plsc.store_scatter(cv, [_i16(2), rows], jnp.bitwise_and(sx, 7))
      return 0
    lax.fori_loop(0, KV, kb, 0)

    pltpu.sync_copy(hv, sh_hist.at[shard])

  base = wid * SHARD
  run_pass(wid, True, cbuf, cp_own)
  pending.append(pltpu.async_copy(w0v, win0.at[pl.ds(base, SHARD)], semO))
  pending.append(pltpu.async_copy(w1v, win1.at[pl.ds(base, SHARD)], semO))
  pending.append(pltpu.async_copy(c0v, cin0.at[:, pl.ds(base, SHARD)], semO))
  pending.append(pltpu.async_copy(c1v, cin1.at[:, pl.ds(base, SHARD)], semO))
  pending.append(pltpu.async_copy(c4v, coorsb.at[:, pl.ds(base, SHARD)], semO))
  run_pass(mirror, False, cbufB, cp_mir)

  plsc.subcore_barrier()

  @pl.when(s < NCOLBLK)
  def _phase_b():
    pltpu.sync_copy(sh_hist.at[:, 0, pl.ds(s * COLS, COLS)], tmp)
    nj = COLS // 16

    def pw(w, acc):
      wv16 = jnp.full((16,), w, jnp.int32)
      new = []
      for j in range(nj):
        cidx = j * 16 + iota
        plsc.store_scatter(ptmp, [wv16, cidx], acc[j])
        v = plsc.load_gather(tmp, [wv16, cidx])
        new.append(acc[j] + v)
      return tuple(new)
    acc = lax.fori_loop(0, NW, pw,
                        tuple(jnp.zeros((16,), jnp.int32) for _ in range(nj)))
    for j in range(nj):
      plsc.store_scatter(ptmp, [_i16(NW), j * 16 + iota], acc[j])
    pltpu.sync_copy(ptmp, sh_pref.at[:, 0, pl.ds(s * COLS, COLS)])

  plsc.subcore_barrier()

  pltpu.sync_copy(sh_pref.at[wid], basev)
  pltpu.sync_copy(sh_pref.at[NW], totv)

  def fb(k, _):
    rows = k * 16 + iota
    w0 = plsc.load_gather(w0v, [rows])
    w1 = plsc.load_gather(w1v, [rows])
    one = _i16(1)
    for w, rv, dv, boff in ((w0, r0v, d0v, 0), (w1, r1v, d1v, NBINS)):
      wb = w + boff
      bse = plsc.load_gather(basev, [zero16, wb])
      r = plsc.load_gather(rv, [rows])
      plsc.store_scatter(rv, [rows], bse + r)
      nb = plsc.load_gather(totv, [zero16, wb])
      dl = jnp.where(nb >= 30, one, zero16) + jnp.where(nb >= 60, one, zero16)
      plsc.store_scatter(dv, [rows], dl)
    return 0
  lax.fori_loop(0, KV, fb, 0)

  pending.append(pltpu.async_copy(r0v, iw0.at[pl.ds(base, SHARD)], semO))
  pending.append(pltpu.async_copy(d0v, dl0.at[pl.ds(base, SHARD)], semO))
  pending.append(pltpu.async_copy(r1v, iw1.at[pl.ds(base, SHARD)], semO))
  pending.append(pltpu.async_copy(d1v, dl1.at[pl.ds(base, SHARD)], semO))
  for cp in pending:
    cp.wait()


def _make_sc():
  mesh = plsc.VectorSubcoreMesh(core_axis_name="c", subcore_axis_name="s",
                                num_cores=NC, num_subcores=NS)
  i32 = jnp.int32
  return pl.kernel(
      _sc_body,
      out_type=(
          jax.ShapeDtypeStruct((N,), i32),
          jax.ShapeDtypeStruct((3, N), i32),
          jax.ShapeDtypeStruct((N,), i32),
          jax.ShapeDtypeStruct((3, N), i32),
          jax.ShapeDtypeStruct((N,), i32),
          jax.ShapeDtypeStruct((N,), i32),
          jax.ShapeDtypeStruct((N,), i32),
          jax.ShapeDtypeStruct((N,), i32),
          jax.ShapeDtypeStruct((4, N), i32),
      ),
      mesh=mesh,
      compiler_params=pltpu.CompilerParams(needs_layout_passes=False,
                                           skip_device_barrier=True),
      scratch_types=(
          pltpu.VMEM((4, SHARD), i32),
          pltpu.VMEM((4, SHARD), i32),
          pltpu.VMEM((SHARD,), i32),
          pltpu.VMEM((SHARD,), i32),
          pltpu.VMEM((SHARD,), i32),
          pltpu.VMEM((SHARD,), i32),
          pltpu.VMEM((SHARD,), i32),
          pltpu.VMEM((SHARD,), i32),
          pltpu.VMEM((3, SHARD), i32),
          pltpu.VMEM((3, SHARD), i32),
          pltpu.VMEM((4, SHARD), i32),
          pltpu.VMEM((1, NB2), i32),
          pltpu.VMEM((NW, COLS), i32),
          pltpu.VMEM((NW + 1, COLS), i32),
          pltpu.VMEM((1, NB2), i32),
          pltpu.VMEM((1, NB2), i32),
          pltpu.SemaphoreType.DMA,
          pltpu.SemaphoreType.DMA,
          pltpu.SemaphoreType.DMA,
          pltpu.VMEM_SHARED((NW, 1, NB2), i32),
          pltpu.VMEM_SHARED((NW + 1, 1, NB2), i32),
      ),
  )


def _pos_table():
  pos_length = 42
  i = np.arange(pos_length, dtype=np.float64)
  inv_freq = 10000.0 ** (2 * np.floor(i / 2) / pos_length)
  v = np.arange(8, dtype=np.float64) - 4.0
  e = v[:, None] / inv_freq[None, :]
  tab = np.where(i[None, :] % 2 == 0, np.sin(e), np.cos(e))
  T24 = np.zeros((24, 128), dtype=np.float32)
  T24[0:8, 0:42] = tab
  T24[8:16, 42:84] = tab
  T24[16:24, 84:126] = tab
  T = np.zeros((48, 256), dtype=np.float32)
  T[0:24, 0:128] = T24
  T[24:48, 128:256] = T24
  return T


_T_NP = _pos_table()


def _tc_body(cref, tref, p0ref, p1ref):
  c4 = cref[...]
  rows = c4.shape[1]
  z4 = c4[1:2, :]
  y4 = c4[2:3, :]
  x4 = c4[3:4, :]
  one = jnp.int32(1)
  packs = []
  for off in (8, 4):
    packs.append(
        jnp.left_shift(one, jnp.bitwise_and(x4 + off, 7))
        | jnp.left_shift(one, jnp.bitwise_and(y4 + off, 7) + 8)
        | jnp.left_shift(one, jnp.bitwise_and(z4 + off, 7) + 16))
  pt = jnp.concatenate(packs, axis=0).T
  p0c = pt[:, 0:1]
  p1c = pt[:, 1:2]
  lane = lax.broadcasted_iota(jnp.int32, (rows, 48), 1)
  lanemod = jnp.where(lane < 24, lane, lane - 24)
  psel = jnp.where(lane < 24, p0c, p1c)
  oh = jnp.bitwise_and(jnp.right_shift(psel, lanemod), 1).astype(jnp.float32)
  big = jnp.dot(oh, tref[...], preferred_element_type=jnp.float32)
  p0ref[...] = big[:, 0:128]
  p1ref[...] = big[:, 128:256]


def _make_tc():
  blk = 8192
  grid = N // blk
  return pl.pallas_call(
      _tc_body,
      grid=(grid,),
      in_specs=[pl.BlockSpec((4, blk), lambda g: (0, g)),
                pl.BlockSpec((48, 256), lambda g: (0, 0))],
      out_specs=[pl.BlockSpec((blk, 128), lambda g: (g, 0)),
                 pl.BlockSpec((blk, 128), lambda g: (g, 0))],
      out_shape=[jax.ShapeDtypeStruct((N, 128), jnp.float32),
                 jax.ShapeDtypeStruct((N, 128), jnp.float32)],
  )


def kernel(voxel_feat, voxel_coords):
  coors = voxel_coords.astype(jnp.int32)
  coorsT = coors.T
  (win0, cin0c, win1, cin1c, iw0, dl0, iw1, dl1,
   coorsb) = _make_sc()(coorsT)
  pos0, pos1 = _make_tc()(coorsT, jnp.asarray(_T_NP))
  used = jnp.arange(N, dtype=jnp.int32)
  return (coorsb.T, voxel_feat, win0, cin0c.T, win1,
          cin1c.T, used, dl0, iw0, dl1, iw1, pos0, pos1)


# speedup vs baseline: 1.0009x; 1.0009x over previous
import numpy as np
import jax
import jax.numpy as jnp
from jax import lax
from jax.experimental import pallas as pl
from jax.experimental.pallas import tpu as pltpu
from jax.experimental.pallas import tpu_sc as plsc

N = 32768
NBINS = 768
NB2 = 2 * NBINS
NC, NS = 2, 16
NW = NC * NS
SHARD = N // NW
KV = SHARD // 16
COLS = 128
NCOLBLK = NB2 // COLS


def _i16(v):
  return jnp.full((16,), v, jnp.int32)


def _sc_body(coords, win0, cin0, win1, cin1, iw0, dl0, iw1, dl1, coorsb,
             cbuf, cbufB, w0v, w1v, r0v, r1v, d0v, d1v, c0v, c1v, c4v, hv,
             tmp, ptmp, basev, totv, semA, semB, semO, sh_hist, sh_pref):
  c = lax.axis_index("c")
  s = lax.axis_index("s")
  wid = c * NS + s
  mirror = (1 - c) * NS + s
  iota = lax.iota(jnp.int32, 16)
  zero16 = jnp.zeros((16,), jnp.int32)

  cp_own = pltpu.async_copy(coords.at[:, pl.ds(wid * SHARD, SHARD)],
                            cbuf, semA)
  cp_mir = pltpu.async_copy(coords.at[:, pl.ds(mirror * SHARD, SHARD)],
                            cbufB, semB)
  pending = []

  def run_pass(shard, is_own, buf, cp):
    def zb(j, _):
      plsc.store_scatter(hv, [zero16, j * 16 + iota], zero16)
      return 0
    lax.fori_loop(0, NB2 // 16, zb, 0)

    cp.wait()

    def kb(k, _):
      rows = k * 16 + iota
      b = plsc.load_gather(buf, [_i16(0), rows])
      z = plsc.load_gather(buf, [_i16(1), rows])
      y = plsc.load_gather(buf, [_i16(2), rows])
      x = plsc.load_gather(buf, [_i16(3), rows])
      if is_own:
        plsc.store_scatter(c4v, [_i16(0), rows], b)
        plsc.store_scatter(c4v, [_i16(1), rows], z)
        plsc.store_scatter(c4v, [_i16(2), rows], y)
        plsc.store_scatter(c4v, [_i16(3), rows], x)
      for off, boff, wv, rv, cv in ((8, 0, w0v, r0v, c0v),
                                    (4, NBINS, w1v, r1v, c1v)):
        sz = z + off
        sy = y + off
        sx = x + off
        w = (b * 48 + jnp.right_shift(sx, 3) * 12
             + jnp.right_shift(sy, 3) * 3 + jnp.right_shift(sz, 3))
        cnt, last = plsc.scan_count(w)
        g = plsc.load_gather(hv, [zero16, w + boff])
        plsc.store_scatter(hv, [zero16, w + boff], g + cnt, mask=last)
        if is_own:
          plsc.store_scatter(wv, [rows], w)
          plsc.store_scatter(rv, [rows], g + cnt - 1)
          plsc.store_scatter(cv, [_i16(0), rows], jnp.bitwise_and(sz, 7))
          plsc.store_scatter(cv, [_i16(1), rows], jnp.bitwise_and(sy, 7))
          plsc.store_scatter(cv, [_i16(2), rows], jnp.bitwise_and(sx, 7))
      return 0
    lax.fori_loop(0, KV, kb, 0)

    pltpu.sync_copy(hv, sh_hist.at[shard])

  base = wid * SHARD
  run_pass(wid, True, cbuf, cp_own)
  pending.append(pltpu.async_copy(w0v, win0.at[pl.ds(base, SHARD)], semO))
  pending.append(pltpu.async_copy(w1v, win1.at[pl.ds(base, SHARD)], semO))
  pending.append(pltpu.async_copy(c0v, cin0.at[:, pl.ds(base, SHARD)], semO))
  pending.append(pltpu.async_copy(c1v, cin1.at[:, pl.ds(base, SHARD)], semO))
  pending.append(pltpu.async_copy(c4v, coorsb.at[:, pl.ds(base, SHARD)], semO))
  run_pass(mirror, False, cbufB, cp_mir)

  plsc.subcore_barrier()

  @pl.when(s < NCOLBLK)
  def _phase_b():
    pltpu.sync_copy(sh_hist.at[:, 0, pl.ds(s * COLS, COLS)], tmp)
    nj = COLS // 16

    def pw(w, acc):
      wv16 = jnp.full((16,), w, jnp.int32)
      new = []
      for j in range(nj):
        cidx = j * 16 + iota
        plsc.store_scatter(ptmp, [wv16, cidx], acc[j])
        v = plsc.load_gather(tmp, [wv16, cidx])
        new.append(acc[j] + v)
      return tuple(new)
    acc = lax.fori_loop(0, NW, pw,
                        tuple(jnp.zeros((16,), jnp.int32) for _ in range(nj)))
    for j in range(nj):
      plsc.store_scatter(ptmp, [_i16(NW), j * 16 + iota], acc[j])
    pltpu.sync_copy(ptmp, sh_pref.at[:, 0, pl.ds(s * COLS, COLS)])

  plsc.subcore_barrier()

  pltpu.sync_copy(sh_pref.at[wid], basev)
  pltpu.sync_copy(sh_pref.at[NW], totv)

  def fb(k, _):
    rows = k * 16 + iota
    w0 = plsc.load_gather(w0v, [rows])
    w1 = plsc.load_gather(w1v, [rows])
    one = _i16(1)
    for w, rv, dv, boff in ((w0, r0v, d0v, 0), (w1, r1v, d1v, NBINS)):
      wb = w + boff
      bse = plsc.load_gather(basev, [zero16, wb])
      r = plsc.load_gather(rv, [rows])
      plsc.store_scatter(rv, [rows], bse + r)
      nb = plsc.load_gather(totv, [zero16, wb])
      dl = jnp.where(nb >= 30, one, zero16) + jnp.where(nb >= 60, one, zero16)
      plsc.store_scatter(dv, [rows], dl)
    return 0
  lax.fori_loop(0, KV, fb, 0)

  pending.append(pltpu.async_copy(r0v, iw0.at[pl.ds(base, SHARD)], semO))
  pending.append(pltpu.async_copy(d0v, dl0.at[pl.ds(base, SHARD)], semO))
  pending.append(pltpu.async_copy(r1v, iw1.at[pl.ds(base, SHARD)], semO))
  pending.append(pltpu.async_copy(d1v, dl1.at[pl.ds(base, SHARD)], semO))
  for cp in pending:
    cp.wait()


def _make_sc():
  mesh = plsc.VectorSubcoreMesh(core_axis_name="c", subcore_axis_name="s",
                                num_cores=NC, num_subcores=NS)
  i32 = jnp.int32
  return pl.kernel(
      _sc_body,
      out_type=(
          jax.ShapeDtypeStruct((N,), i32),
          jax.ShapeDtypeStruct((3, N), i32),
          jax.ShapeDtypeStruct((N,), i32),
          jax.ShapeDtypeStruct((3, N), i32),
          jax.ShapeDtypeStruct((N,), i32),
          jax.ShapeDtypeStruct((N,), i32),
          jax.ShapeDtypeStruct((N,), i32),
          jax.ShapeDtypeStruct((N,), i32),
          jax.ShapeDtypeStruct((4, N), i32),
      ),
      mesh=mesh,
      compiler_params=pltpu.CompilerParams(needs_layout_passes=False,
                                           skip_device_barrier=True),
      scratch_types=(
          pltpu.VMEM((4, SHARD), i32),
          pltpu.VMEM((4, SHARD), i32),
          pltpu.VMEM((SHARD,), i32),
          pltpu.VMEM((SHARD,), i32),
          pltpu.VMEM((SHARD,), i32),
          pltpu.VMEM((SHARD,), i32),
          pltpu.VMEM((SHARD,), i32),
          pltpu.VMEM((SHARD,), i32),
          pltpu.VMEM((3, SHARD), i32),
          pltpu.VMEM((3, SHARD), i32),
          pltpu.VMEM((4, SHARD), i32),
          pltpu.VMEM((1, NB2), i32),
          pltpu.VMEM((NW, COLS), i32),
          pltpu.VMEM((NW + 1, COLS), i32),
          pltpu.VMEM((1, NB2), i32),
          pltpu.VMEM((1, NB2), i32),
          pltpu.SemaphoreType.DMA,
          pltpu.SemaphoreType.DMA,
          pltpu.SemaphoreType.DMA,
          pltpu.VMEM_SHARED((NW, 1, NB2), i32),
          pltpu.VMEM_SHARED((NW + 1, 1, NB2), i32),
      ),
  )


def _pos_table():
  pos_length = 42
  i = np.arange(pos_length, dtype=np.float64)
  inv_freq = 10000.0 ** (2 * np.floor(i / 2) / pos_length)
  v = np.arange(8, dtype=np.float64) - 4.0
  e = v[:, None] / inv_freq[None, :]
  tab = np.where(i[None, :] % 2 == 0, np.sin(e), np.cos(e))
  T24 = np.zeros((24, 128), dtype=np.float32)
  T24[0:8, 0:42] = tab
  T24[8:16, 42:84] = tab
  T24[16:24, 84:126] = tab
  T = np.zeros((48, 256), dtype=np.float32)
  T[0:24, 0:128] = T24
  T[24:48, 128:256] = T24
  return T


_T_NP = _pos_table()


def _tc_body(cref, tref, p0ref, p1ref):
  c4 = cref[...]
  rows = c4.shape[1]
  z4 = c4[1:2, :]
  y4 = c4[2:3, :]
  x4 = c4[3:4, :]
  one = jnp.int32(1)
  packs = []
  for off in (8, 4):
    packs.append(
        jnp.left_shift(one, jnp.bitwise_and(x4 + off, 7))
        | jnp.left_shift(one, jnp.bitwise_and(y4 + off, 7) + 8)
        | jnp.left_shift(one, jnp.bitwise_and(z4 + off, 7) + 16))
  pt = jnp.concatenate(packs, axis=0).T
  p0c = pt[:, 0:1]
  p1c = pt[:, 1:2]
  lane = lax.broadcasted_iota(jnp.int32, (rows, 48), 1)
  lanemod = jnp.where(lane < 24, lane, lane - 24)
  psel = jnp.where(lane < 24, p0c, p1c)
  oh = jnp.bitwise_and(jnp.right_shift(psel, lanemod), 1).astype(jnp.float32)
  big = jnp.dot(oh, tref[...], preferred_element_type=jnp.float32)
  p0ref[...] = big[:, 0:128]
  p1ref[...] = big[:, 128:256]


def _make_tc():
  blk = 8192
  grid = N // blk
  return pl.pallas_call(
      _tc_body,
      grid=(grid,),
      in_specs=[pl.BlockSpec((4, blk), lambda g: (0, g)),
                pl.BlockSpec((48, 256), lambda g: (0, 0))],
      out_specs=[pl.BlockSpec((blk, 128), lambda g: (g, 0)),
                 pl.BlockSpec((blk, 128), lambda g: (g, 0))],
      out_shape=[jax.ShapeDtypeStruct((N, 128), jnp.float32),
                 jax.ShapeDtypeStruct((N, 128), jnp.float32)],
      compiler_params=pltpu.CompilerParams(
          dimension_semantics=("parallel",)),
  )


def kernel(voxel_feat, voxel_coords):
  coors = voxel_coords.astype(jnp.int32)
  coorsT = coors.T
  (win0, cin0c, win1, cin1c, iw0, dl0, iw1, dl1,
   coorsb) = _make_sc()(coorsT)
  pos0, pos1 = _make_tc()(coorsT, jnp.asarray(_T_NP))
  used = jnp.arange(N, dtype=jnp.int32)
  return (coorsb.T, voxel_feat, win0, cin0c.T, win1,
          cin1c.T, used, dl0, iw0, dl1, iw1, pos0, pos1)
